# Initial kernel scaffold; baseline (speedup 1.0000x reference)
import jax
import jax.numpy as jnp
from jax.experimental import pallas as pl


def kernel(x, edge_index, batch, group, W1, b1, W2, b2, Wg, bg, Wf, bf):
    raise NotImplementedError("write your pallas kernel here")


# speedup vs baseline: 9.9036x; 9.9036x over previous
import functools

import jax
import jax.numpy as jnp
from jax import lax
from jax.experimental import pallas as pl
from jax.experimental.pallas import tpu as pltpu
from jax.experimental.pallas import tpu_sc as plsc

N = 10000
E = 160000
F_IN = 256
H = 256
D = 128
G = 16
FAM = 64
B = 512

NC = 2
NS = 16
K = 128

NPAD = 10112
RZ = NPAD // NS
RO = 632
RO_LAST = N - 15 * RO
CHD = 40
EPD = NC * NS * CHD * K
CHP = 79
EPP = NS * CHP * K


def _deg_body(dstR, z16, out, idx_v, ones_v, acc, _):
    c = lax.axis_index("c")
    s = lax.axis_index("s")
    pltpu.sync_copy(z16.at[pl.ds(s * RZ, RZ)], acc.at[pl.ds(s * RZ, RZ)])

    def fill(i, carry):
        ones_v[i, :] = jnp.full((16,), 1.0, jnp.float32)
        return carry
    lax.fori_loop(0, K, fill, 0)

    pltpu.sync_copy(dstR.at[c, s], idx_v)
    plsc.subcore_barrier()

    def chunk(j, carry):
        pltpu.sync_copy(ones_v, acc.at[idx_v.at[j]], add=True)
        return carry
    lax.fori_loop(0, CHD, chunk, 0)

    plsc.subcore_barrier()
    _copy_out_rows(acc, out, c, s)


def _copy_out_rows(acc, out, c, s):
    @pl.when(s < NS - 1)
    def _():
        pltpu.sync_copy(acc.at[pl.ds(s * RO, RO)],
                        out.at[c, pl.ds(s * RO, RO)])

    @pl.when(s == NS - 1)
    def _():
        pltpu.sync_copy(acc.at[pl.ds((NS - 1) * RO, RO_LAST)],
                        out.at[c, pl.ds((NS - 1) * RO, RO_LAST)])


@functools.lru_cache(maxsize=None)
def _sc_calls():
    mesh = plsc.VectorSubcoreMesh(
        core_axis_name="c", subcore_axis_name="s",
        num_cores=NC, num_subcores=NS)
    cp = pltpu.CompilerParams(use_tc_tiling_on_sc=False)
    deg_call = pl.kernel(
        _deg_body,
        out_type=jax.ShapeDtypeStruct((NC, N, 16), jnp.float32),
        mesh=mesh,
        scratch_types=[
            pltpu.VMEM((CHD, K), jnp.int32),
            pltpu.VMEM((K, 16), jnp.float32),
            pltpu.VMEM_SHARED((NPAD, 16), jnp.float32),
            pltpu.SemaphoreType.DMA,
        ],
        compiler_params=cp,
    )

    def make_prop(dh):
        return pl.kernel(
            _prop_body,
            out_type=jax.ShapeDtypeStruct((NC, N, dh), jnp.float32),
            mesh=mesh,
            scratch_types=[
                pltpu.VMEM((CHP, K), jnp.int32),
                pltpu.VMEM((CHP, K), jnp.int32),
                pltpu.VMEM((K, dh), jnp.float32),
                pltpu.VMEM_SHARED((NPAD, dh), jnp.float32),
                pltpu.SemaphoreType.DMA,
            ],
            compiler_params=cp,
        )
    return deg_call, make_prop(128), make_prop(64)


def _prop_body(srcR, dstR, table, zpad, out, src_v, dst_v, rows_v, acc, sem):
    c = lax.axis_index("c")
    s = lax.axis_index("s")
    pltpu.sync_copy(zpad.at[pl.ds(s * RZ, RZ)], acc.at[pl.ds(s * RZ, RZ)])
    pltpu.sync_copy(srcR.at[c, s], src_v)
    pltpu.sync_copy(dstR.at[s], dst_v)
    plsc.subcore_barrier()

    def chunk(j, carry):
        pltpu.async_copy(table.at[src_v.at[j]], rows_v, sem).wait()
        pltpu.sync_copy(rows_v, acc.at[dst_v.at[j]], add=True)
        return carry
    lax.fori_loop(0, CHP, chunk, 0)

    plsc.subcore_barrier()
    _copy_out_rows(acc, out, c, s)


def _dinv_from(deg16_ref):
    d = deg16_ref[0] + deg16_ref[1]
    deg = jnp.sum(d, axis=1, keepdims=True) * (1.0 / 16.0) + 1.0
    return lax.rsqrt(deg)


def _a_body(x_ref, w1_ref, deg16_ref, out_ref):
    dinv = _dinv_from(deg16_ref)
    g = jnp.dot(x_ref[...], w1_ref[...], preferred_element_type=jnp.float32)
    g = g * dinv
    out_ref[0] = g[:, :128]
    out_ref[1] = g[:, 128:]


def _c_body(acc_ref, g_ref, deg16_ref, b1_ref, w2_ref, out_ref):
    dinv = _dinv_from(deg16_ref)
    b1 = b1_ref[...]
    ha = dinv * (acc_ref[0] + g_ref[0]) + b1[:128][None, :]
    hb = dinv * (acc_ref[1] + g_ref[1]) + b1[128:][None, :]
    h1 = jnp.maximum(jnp.concatenate([ha, hb], axis=1), 0.0)
    g2 = jnp.dot(h1, w2_ref[...], preferred_element_type=jnp.float32) * dinv
    out_ref[0] = g2[:, :64]
    out_ref[1] = g2[:, 64:]


def _d_body(acc_ref, g_ref, deg16_ref, b2_ref, batch_ref, wg_ref, bg_ref,
            wf_ref, bf_ref, emb_ref, gl_ref, fl_ref):
    dinv = _dinv_from(deg16_ref)
    ha = dinv * (acc_ref[0] + g_ref[0])
    hb = dinv * (acc_ref[1] + g_ref[1])
    h2 = jnp.concatenate([ha, hb], axis=1) + b2_ref[...][None, :]

    rblk = 1000
    sums = jnp.zeros((B, D), jnp.float32)
    cnt = jnp.zeros((B, 1), jnp.float32)
    for i in range(N // rblk):
        bb = batch_ref[i]
        oh = (bb[None, :] == lax.broadcasted_iota(jnp.int32, (B, rblk), 0))
        oh = oh.astype(jnp.float32)
        sums = sums + jnp.dot(oh, h2[i * rblk:(i + 1) * rblk, :],
                              preferred_element_type=jnp.float32)
        cnt = cnt + jnp.sum(oh, axis=1, keepdims=True)

    emb = sums / jnp.maximum(cnt, 1.0)
    gl = jnp.dot(emb, wg_ref[...], preferred_element_type=jnp.float32) \
        + bg_ref[...][None, :]
    m = jnp.max(gl, axis=1, keepdims=True)
    eqm = gl == m
    gi = lax.broadcasted_iota(jnp.int32, (B, G), 1)
    gid = jnp.min(jnp.where(eqm, gi, G), axis=1, keepdims=True)
    fl = jnp.zeros((B, FAM), jnp.float32)
    for g_ in range(G):
        sel = (gid == g_).astype(jnp.float32)
        head = jnp.dot(emb, wf_ref[g_], preferred_element_type=jnp.float32) \
            + bf_ref[g_][None, :]
        fl = fl + sel * head
    emb_ref[...] = emb
    gl_ref[...] = gl
    fl_ref[...] = fl


def kernel(x, edge_index, batch, group, W1, b1, W2, b2, Wg, bg, Wf, bf):
    src = edge_index[0].astype(jnp.int32)
    dst = edge_index[1].astype(jnp.int32)

    pad_d = EPD - E
    dst_deg = jnp.concatenate(
        [dst, N + (jnp.arange(pad_d, dtype=jnp.int32) % 16)])
    dstR_deg = dst_deg.reshape(NC, NS, CHD, K)

    pad_p = EPP - E
    srcp = jnp.concatenate([src, jnp.zeros((pad_p,), jnp.int32)])
    dstp = jnp.concatenate(
        [dst, N + (jnp.arange(pad_p, dtype=jnp.int32) % 16)])
    srcR = jnp.stack([srcp, srcp + N]).reshape(NC, NS, CHP, K)
    dstR = dstp.reshape(NS, CHP, K)

    z128 = jnp.zeros((NPAD, 128), jnp.float32)
    z64 = jnp.zeros((NPAD, 64), jnp.float32)
    z16 = jnp.zeros((NPAD, 16), jnp.float32)
    batch2 = batch.astype(jnp.int32).reshape(N // 1000, 1000)

    deg_call, prop128, prop64 = _sc_calls()
    deg16 = deg_call(dstR_deg, z16)

    g1 = pl.pallas_call(
        _a_body,
        out_shape=jax.ShapeDtypeStruct((NC, N, 128), jnp.float32),
    )(x, W1, deg16)

    acc1 = prop128(srcR, dstR, g1.reshape(NC * N, 128), z128)

    g2 = pl.pallas_call(
        _c_body,
        out_shape=jax.ShapeDtypeStruct((NC, N, 64), jnp.float32),
    )(acc1, g1, deg16, b1, W2)

    acc2 = prop64(srcR, dstR, g2.reshape(NC * N, 64), z64)

    emb, gl, fl = pl.pallas_call(
        _d_body,
        out_shape=(
            jax.ShapeDtypeStruct((B, D), jnp.float32),
            jax.ShapeDtypeStruct((B, G), jnp.float32),
            jax.ShapeDtypeStruct((B, FAM), jnp.float32),
        ),
    )(acc2, g2, deg16, b2, batch2, Wg, bg, Wf, bf)
    return (emb, gl, fl)
